# Initial kernel scaffold; baseline (speedup 1.0000x reference)
import jax
import jax.numpy as jnp
from jax.experimental import pallas as pl


def kernel(student_id, exercise_id, S, theta_tuda, theta, diff_table, disc_table):
    raise NotImplementedError("write your pallas kernel here")


# speedup vs baseline: 1.1419x; 1.1419x over previous
import functools

import jax
import jax.numpy as jnp
from jax import lax
from jax.experimental import pallas as pl
from jax.experimental.pallas import tpu as pltpu
from jax.experimental.pallas import tpu_sc as plsc

B = 16384
K = 128
NC = 2
NS = 16
NW = NC * NS
ROWS_PER_W = B // NW
CH = 128
NCH = ROWS_PER_W // CH


def _sc_body(sid_hbm, eid_hbm, s_hbm, theta_hbm, diff_hbm, disc_hbm,
             student_out, diff_out, disc_out,
             sid_v, eid_v, theta_v, disc_v, s_buf, diff_buf,
             sem_a, sem_b, sem_c):
    wid = lax.axis_index("s") * NC + lax.axis_index("c")
    cbase = wid * NCH

    pltpu.sync_copy(sid_hbm.at[pl.ds(cbase, NCH)], sid_v)
    pltpu.sync_copy(eid_hbm.at[pl.ds(cbase, NCH)], eid_v)

    handles = []
    for j in range(NCH):
        handles.append(
            pltpu.async_copy(theta_hbm.at[sid_v.at[j]],
                             theta_v.at[pl.ds(j * CH, CH)], sem_a))
        handles.append(
            pltpu.async_copy(disc_hbm.at[eid_v.at[j]], disc_v.at[j], sem_b))
    for h in handles:
        h.wait()
    pltpu.sync_copy(disc_v, disc_out.at[pl.ds(cbase, NCH)])

    for j in range(NCH):
        chunk = cbase + j
        rbase = chunk * CH
        h_diff = pltpu.async_copy(diff_hbm.at[eid_v.at[j]], diff_buf, sem_a)
        pltpu.sync_copy(s_hbm.at[pl.ds(rbase, CH)], s_buf)
        h_diff.wait()
        h_dout = pltpu.async_copy(diff_buf, diff_out.at[pl.ds(rbase, CH)],
                                  sem_b)

        def group_body(g, _):
            tv = theta_v[pl.ds(j * CH + g * 16, 16)]
            for l in range(16):
                t = tv.at[jnp.full((16,), l, dtype=jnp.int32)].get(
                    mode="promise_in_bounds")
                one_m_t = 1.0 - t
                r = g * 16 + l
                for c in range(K // 16):
                    sl = pl.ds(c * 16, 16)
                    s_buf[r, sl] = s_buf[r, sl] * one_m_t + t
            return 0

        lax.fori_loop(0, CH // 16, group_body, 0)
        h_dout.wait()
        pltpu.async_copy(s_buf, student_out.at[pl.ds(rbase, CH)],
                         sem_c).wait()


@jax.jit
def _run(sid2, eid2, S, theta_flat, diff_table, disc_flat):
    mesh = plsc.VectorSubcoreMesh(core_axis_name="c", subcore_axis_name="s")
    f = pl.kernel(
        _sc_body,
        out_type=[
            jax.ShapeDtypeStruct((B, K), jnp.float32),
            jax.ShapeDtypeStruct((B, K), jnp.float32),
            jax.ShapeDtypeStruct((B // CH, CH), jnp.float32)
        ],
        mesh=mesh,
        scratch_types=[
            pltpu.VMEM((NCH, CH), jnp.int32),
            pltpu.VMEM((NCH, CH), jnp.int32),
            pltpu.VMEM((ROWS_PER_W,), jnp.float32),
            pltpu.VMEM((NCH, CH), jnp.float32),
            pltpu.VMEM((CH, K), jnp.float32),
            pltpu.VMEM((CH, K), jnp.float32),
            pltpu.SemaphoreType.DMA,
            pltpu.SemaphoreType.DMA,
            pltpu.SemaphoreType.DMA,
        ],
    )
    return f(sid2, eid2, S, theta_flat, diff_table, disc_flat)


def kernel(student_id, exercise_id, S, theta_tuda, theta, diff_table,
           disc_table):
    sid2 = student_id.reshape(B // CH, CH)
    eid2 = exercise_id.reshape(B // CH, CH)
    theta_flat = theta.reshape(-1)
    disc_flat = disc_table.reshape(-1)
    student_ts, diff_ts, disc_rows = _run(sid2, eid2, S, theta_flat,
                                          diff_table, disc_flat)
    return student_ts, diff_ts, disc_rows.reshape(B, 1)


# speedup vs baseline: 2.7414x; 2.4007x over previous
import functools

import jax
import jax.numpy as jnp
from jax import lax
from jax.experimental import pallas as pl
from jax.experimental.pallas import tpu as pltpu
from jax.experimental.pallas import tpu_sc as plsc

B = 16384
K = 128
NC = 2
NS = 16
NW = NC * NS
ROWS_PER_W = B // NW
CH = 128
NCH = ROWS_PER_W // CH


def _sc_body(sid_hbm, eid_hbm, s_hbm, theta_hbm, diff_hbm, disc_hbm,
             student_out, diff_out, disc_out,
             sid_v, eid_v, theta_v, disc_v, s_buf, diff_buf,
             sem_a, sem_b, sem_c):
    wid = lax.axis_index("s") * NC + lax.axis_index("c")
    cbase = wid * NCH

    pltpu.sync_copy(sid_hbm.at[pl.ds(cbase, NCH)], sid_v)
    pltpu.sync_copy(eid_hbm.at[pl.ds(cbase, NCH)], eid_v)

    handles = []
    for j in range(NCH):
        handles.append(
            pltpu.async_copy(theta_hbm.at[sid_v.at[j]],
                             theta_v.at[pl.ds(j * CH, CH)], sem_a))
        handles.append(
            pltpu.async_copy(disc_hbm.at[eid_v.at[j]], disc_v.at[j], sem_b))
    for h in handles:
        h.wait()
    pltpu.sync_copy(disc_v, disc_out.at[pl.ds(cbase, NCH)])

    for j in range(NCH):
        chunk = cbase + j
        rbase = chunk * CH
        h_diff = pltpu.async_copy(diff_hbm.at[eid_v.at[j]], diff_buf, sem_a)
        pltpu.sync_copy(s_hbm.at[pl.ds(rbase, CH)], s_buf)
        h_diff.wait()
        h_dout = pltpu.async_copy(diff_buf, diff_out.at[pl.ds(rbase, CH)],
                                  sem_b)

        def group_body(g, _):
            tv = theta_v[pl.ds(j * CH + g * 16, 16)]
            for l in range(16):
                t = tv.at[jnp.full((16,), l, dtype=jnp.int32)].get(
                    mode="promise_in_bounds")
                one_m_t = 1.0 - t
                r = g * 16 + l
                for c in range(K // 16):
                    sl = pl.ds(c * 16, 16)
                    s_buf[r, sl] = s_buf[r, sl] * one_m_t + t
            return 0

        lax.fori_loop(0, CH // 16, group_body, 0)
        h_dout.wait()
        pltpu.async_copy(s_buf, student_out.at[pl.ds(rbase, CH)],
                         sem_c).wait()


@jax.jit
def _run(sid2, eid2, S, theta_flat, diff_table, disc_flat):
    mesh = plsc.VectorSubcoreMesh(core_axis_name="c", subcore_axis_name="s")
    f = pl.kernel(
        _sc_body,
        out_type=[
            jax.ShapeDtypeStruct((B, K), jnp.float32),
            jax.ShapeDtypeStruct((B, K), jnp.float32),
            jax.ShapeDtypeStruct((B // CH, CH), jnp.float32)
        ],
        mesh=mesh,
        scratch_types=[
            pltpu.VMEM((NCH, CH), jnp.int32),
            pltpu.VMEM((NCH, CH), jnp.int32),
            pltpu.VMEM((ROWS_PER_W,), jnp.float32),
            pltpu.VMEM((NCH, CH), jnp.float32),
            pltpu.VMEM((CH, K), jnp.float32),
            pltpu.VMEM((CH, K), jnp.float32),
            pltpu.SemaphoreType.DMA,
            pltpu.SemaphoreType.DMA,
            pltpu.SemaphoreType.DMA,
        ],
    )
    return f(sid2, eid2, S, theta_flat, diff_table, disc_flat)


NPAD = 1000448


def kernel(student_id, exercise_id, S, theta_tuda, theta, diff_table,
           disc_table):
    sid2 = student_id.reshape(B // CH, CH)
    eid2 = exercise_id.reshape(B // CH, CH)
    n = theta.shape[0]
    theta_flat = jnp.pad(theta, ((0, NPAD - n), (0, 0))).reshape(-1)
    disc_flat = jnp.pad(disc_table, ((0, NPAD - n), (0, 0))).reshape(-1)
    student_ts, diff_ts, disc_rows = _run(sid2, eid2, S, theta_flat,
                                          diff_table, disc_flat)
    return student_ts, diff_ts, disc_rows.reshape(B, 1)
